# baseline (speedup 1.0000x reference)
import numpy as np
import jax
import jax.numpy as jnp
from jax.experimental import pallas as pl
from jax.experimental.pallas import tpu as pltpu

_B, _T, _N, _H = 4, 64, 111, 111
_L = 4
_HEADS = 3
_HD = _N // _HEADS
_G = _B * _T
_GB = 32
_NBLK = _G // _GB
_RB = _GB * _N
_ROWS = _G * _N
_K = 8624
_EPS_BN = 1e-5
_P = 1 + 3 * _L

_f32 = jnp.float32

_B1, _G1, _BE1, _B2, _G2, _BE2 = 0, 1, 2, 3, 4, 5
_EB, _SG, _SBE, _AB = 6, 7, 8, 9
_OB, _LN1G, _LN1B, _M2B, _LN2G, _LN2B = 10, 11, 12, 13, 14, 15


def _dotT(x, w):
    return jax.lax.dot_general(x, w, (((1,), (1,)), ((), ())),
                               preferred_element_type=_f32)


def _dot(x, w):
    return jax.lax.dot_general(x, w, (((1,), (0,)), ((), ())),
                               preferred_element_type=_f32)


def _mega_kernel(a_ref, v_ref, initw_ref, initb_ref, eps_ref,
                 wmat_ref, wrow_ref, m1w_ref, m1b_ref, m2w_ref,
                 wqkv_ref, bqkv_ref, owsl_ref, cw_ref, cb_ref,
                 lat_ref, log_ref, ro_ref,
                 adj_s, h_s, z_s, xr_s, hm_s,
                 s1_s, q1_s, s2_s, q2_s, ro_s, lg_s):
    p = pl.program_id(0)
    i = pl.program_id(1)
    ph = (p - 1) % 3
    is_g1 = (p >= 1) & (ph == 0)
    is_g2 = (p >= 1) & (ph == 1)
    is_g3 = (p >= 1) & (ph == 2)

    def row(j):
        return wrow_ref[0, j:j + 1, :]

    @pl.when(p == 0)
    def _stage_a():
        a = a_ref[...]
        i32 = jax.lax.bitcast_convert_type(a, jnp.int32)
        sign = jnp.int32(-2147483648)
        mask7 = jnp.int32(0x7FFFFFFF)
        km = jnp.where(i32 < 0, i32 ^ mask7, i32)
        pfx = jnp.zeros((_GB, 1, 1), jnp.int32)
        for bit in range(31, -1, -1):
            bitval = jnp.int32(np.int32(np.uint32(1 << bit)))
            cb = pfx | bitval
            c = cb ^ sign
            lt = (km < c).astype(_f32)
            cnt = jnp.sum(jnp.sum(lt, axis=2, keepdims=True),
                          axis=1, keepdims=True)
            pfx = jnp.where(cnt <= float(_K), cb, pfx)
        key = pfx ^ sign
        thr_i = jnp.where(key < 0, key ^ mask7, key)
        thr = jax.lax.bitcast_convert_type(thr_i, _f32)
        adj_s[i] = (a > thr).astype(_f32)
        h_s[i] = _dotT(v_ref[...], initw_ref[...]) + initb_ref[...]

    @pl.when(is_g1)
    def _g1():
        h = h_s[i]
        parts = []
        for g in range(_GB):
            parts.append(_dot(adj_s[i, g], h[g * _N:(g + 1) * _N, :]))
        agg = jnp.concatenate(parts, axis=0) + eps_ref[0, 0, 0] * h
        z = _dotT(agg, wmat_ref[0, 0]) + row(_B1)
        z_s[i] = z
        s = jnp.sum(z, axis=0, keepdims=True)
        q = jnp.sum(z * z, axis=0, keepdims=True)

        @pl.when(i == 0)
        def _init():
            s1_s[...] = s
            q1_s[...] = q

        @pl.when(i != 0)
        def _acc():
            s1_s[...] += s
            q1_s[...] += q

    @pl.when(is_g2)
    def _g2():
        m = s1_s[...] * (1.0 / _ROWS)
        var = q1_s[...] * (1.0 / _ROWS) - m * m
        sc = row(_G1) / jnp.sqrt(var + _EPS_BN)
        sh = row(_BE1) - m * sc
        y = jnp.maximum(z_s[i] * sc + sh, 0.0)
        z2 = _dotT(y, wmat_ref[0, 1]) + row(_B2)
        z_s[i] = z2
        s = jnp.sum(z2, axis=0, keepdims=True)
        q = jnp.sum(z2 * z2, axis=0, keepdims=True)

        @pl.when(i == 0)
        def _init():
            s2_s[...] = s
            q2_s[...] = q

        @pl.when(i != 0)
        def _acc():
            s2_s[...] += s
            q2_s[...] += q

    @pl.when(is_g3)
    def _g3():
        m = s2_s[...] * (1.0 / _ROWS)
        var = q2_s[...] * (1.0 / _ROWS) - m * m
        sc = row(_G2) / jnp.sqrt(var + _EPS_BN)
        sh = row(_BE2) - m * sc
        h = jnp.maximum(z_s[i] * sc + sh, 0.0)
        h_s[i] = h

        rr = jax.lax.broadcasted_iota(jnp.int32, (_N, _N), 0)
        cc = jax.lax.broadcasted_iota(jnp.int32, (_N, _N), 1)
        upper = cc >= rr
        eye = (cc == rr).astype(_f32)
        ones_row = jnp.ones((1, _N), _f32)

        xrs = []
        hms = []
        ro = jnp.zeros((1, 1), _f32)
        for g in range(_GB):
            hg = h[g * _N:(g + 1) * _N, :]
            xrs.append(_dot(ones_row, hg) * (1.0 / _N))
            hms.append(_dotT(ones_row, hg) * (1.0 / _N))
            mi = _dotT(hg, hg)
            mx = jnp.max(mi, axis=1, keepdims=True)
            mi_n = mi * (1.0 / mx)
            diff = jnp.where(upper, mi_n - eye, 0.0)
            ssq = jnp.sum(jnp.sum(diff * diff, axis=1, keepdims=True),
                          axis=0, keepdims=True)
            ro = ro + jnp.sqrt(ssq)
        xr_s[pl.ds(i * _GB, _GB), :] = jnp.concatenate(xrs, axis=0)
        hm_s[pl.ds(i * _GB, _GB), :] = jnp.concatenate(hms, axis=0)

        first = (p == 3) & (i == 0)

        @pl.when(first)
        def _init():
            ro_s[...] = ro

        @pl.when(jnp.logical_not(first))
        def _acc():
            ro_s[...] += ro

        @pl.when(i == _NBLK - 1)
        def _tail():
            xr = xr_s[...]
            x = _dotT(xr, wmat_ref[0, 2]) + row(_EB)
            mm = jnp.mean(x, axis=0, keepdims=True)
            vv = jnp.mean((x - mm) * (x - mm), axis=0, keepdims=True)
            x = (x - mm) / jnp.sqrt(vv + _EPS_BN) * row(_SG) + row(_SBE)
            x = 0.5 * x * (1.0 + jax.lax.erf(x * np.float32(1.0 / np.sqrt(2.0))))
            gatt = jax.nn.sigmoid(_dotT(x, wmat_ref[0, 3]) + row(_AB))
            hro = gatt * hm_s[...]

            scale = np.float32(1.0 / np.sqrt(_HD))

            def _ln(x, g, b):
                mu = jnp.mean(x, axis=1, keepdims=True)
                va = jnp.mean((x - mu) * (x - mu), axis=1, keepdims=True)
                return (x - mu) / jnp.sqrt(va + _EPS_BN) * g + b

            lgs = []
            for b in range(_B):
                xb = hro[b * _T:(b + 1) * _T, :]
                att = jnp.zeros((_T, _N), _f32)
                for hd in range(_HEADS):
                    q = (_dotT(xb, wqkv_ref[hd, 0]) + bqkv_ref[hd, 0]) * scale
                    kk = _dotT(xb, wqkv_ref[3 + hd, 0]) + bqkv_ref[3 + hd, 0]
                    vvh = _dotT(xb, wqkv_ref[6 + hd, 0]) + bqkv_ref[6 + hd, 0]
                    sc = _dotT(q, kk)
                    mx = jnp.max(sc, axis=1, keepdims=True)
                    e = jnp.exp(sc - mx)
                    pa = e / jnp.sum(e, axis=1, keepdims=True)
                    o = _dot(pa, vvh)
                    att = att + jax.lax.dot_general(
                        o, owsl_ref[hd, 0], (((1,), (1,)), ((), ())),
                        preferred_element_type=_f32)
                att = att + row(_OB)
                x1 = _ln(att, row(_LN1G), row(_LN1B))
                x2 = jnp.maximum(_dotT(x1, m1w_ref[0]) + m1b_ref[0], 0.0)
                x2 = _dotT(x2, m2w_ref[0]) + row(_M2B)
                xo = _ln(x1 + x2, row(_LN2G), row(_LN2B))
                lat = jnp.sum(xo, axis=0, keepdims=True)
                lat_ref[0, b:b + 1, :] = lat
                lgs.append(_dotT(lat, cw_ref[0]) + cb_ref[0])
            lgc = jnp.concatenate(lgs, axis=0)

            @pl.when(p == 3)
            def _lg_init():
                lg_s[...] = lgc

            @pl.when(p != 3)
            def _lg_acc():
                lg_s[...] += lgc

            log_ref[...] = lg_s[...]
            ro_ref[...] = ro_s[...]


def kernel(v, a, init_w, init_b, gin_eps, gin_w1, gin_b1, gin_g1, gin_be1,
           gin_w2, gin_b2, gin_g2, gin_be2, sero_ew, sero_eb, sero_g, sero_be,
           sero_aw, sero_ab, tr_inw, tr_inb, tr_ow, tr_ob, tr_ln1g, tr_ln1b,
           tr_ln2g, tr_ln2b, tr_m1w, tr_m1b, tr_m2w, tr_m2b, cls_w, cls_b):
    a3 = a.reshape(_G, _N, _N)
    v2 = v.reshape(_ROWS, _N)

    wmat = jnp.stack([gin_w1, gin_w2, sero_ew, sero_aw], axis=1)
    wrow = jnp.stack([gin_b1, gin_g1, gin_be1, gin_b2, gin_g2, gin_be2,
                      sero_eb, sero_g, sero_be, sero_ab,
                      tr_ob, tr_ln1g, tr_ln1b, tr_m2b, tr_ln2g, tr_ln2b],
                     axis=1)
    wqkv = jnp.stack([tr_inw[:, base + hd * _HD: base + (hd + 1) * _HD, :]
                      for base in (0, _N, 2 * _N) for hd in range(_HEADS)])
    bqkv = jnp.stack([tr_inb[:, base + hd * _HD: base + (hd + 1) * _HD]
                      for base in (0, _N, 2 * _N)
                      for hd in range(_HEADS)])[:, :, None, :]
    owsl = jnp.stack([tr_ow[:, :, hd * _HD:(hd + 1) * _HD]
                      for hd in range(_HEADS)])

    def _lmap(p, i):
        return jnp.clip((p - 1) // 3, 0, _L - 1)

    def im_av(p, i):
        return (jnp.where(p == 0, i, 0), 0, 0)

    def im_v2(p, i):
        return (jnp.where(p == 0, i, 0), 0)

    def im_const2(p, i):
        return (0, 0)

    def im_l3(p, i):
        return (_lmap(p, i), 0, 0)

    def im_l4(p, i):
        return (0, _lmap(p, i), 0, 0)

    def im_l4a(p, i):
        return (_lmap(p, i), 0, 0, 0)

    in_specs = [
        pl.BlockSpec((_GB, _N, _N), im_av),
        pl.BlockSpec((_RB, _N), im_v2),
        pl.BlockSpec((_N, _N), im_const2),
        pl.BlockSpec((1, _N), im_const2),
        pl.BlockSpec((1, 1, 1), im_l3),
        pl.BlockSpec((1, 4, _N, _N), im_l4a),
        pl.BlockSpec((1, 16, _N), im_l3),
        pl.BlockSpec((1, 2 * _H, _N), im_l3),
        pl.BlockSpec((1, 1, 2 * _H), im_l3),
        pl.BlockSpec((1, _N, 2 * _H), im_l3),
        pl.BlockSpec((9, 1, _HD, _N), im_l4),
        pl.BlockSpec((9, 1, 1, _HD), im_l4),
        pl.BlockSpec((3, 1, _N, _HD), im_l4),
        pl.BlockSpec((1, 2, _N), im_l3),
        pl.BlockSpec((1, 1, 2), im_l3),
    ]
    out_specs = [
        pl.BlockSpec((1, _B, _N), lambda p, i: (_lmap(p, i), 0, 0)),
        pl.BlockSpec((_B, 2), im_const2),
        pl.BlockSpec((1, 1), im_const2),
    ]
    out_shape = [
        jax.ShapeDtypeStruct((_L, _B, _N), _f32),
        jax.ShapeDtypeStruct((_B, 2), _f32),
        jax.ShapeDtypeStruct((1, 1), _f32),
    ]
    scratch_shapes = [
        pltpu.VMEM((_NBLK, _GB, _N, _N), _f32),
        pltpu.VMEM((_NBLK, _RB, _N), _f32),
        pltpu.VMEM((_NBLK, _RB, _N), _f32),
        pltpu.VMEM((_G, _N), _f32),
        pltpu.VMEM((_G, _N), _f32),
        pltpu.VMEM((1, _N), _f32),
        pltpu.VMEM((1, _N), _f32),
        pltpu.VMEM((1, _N), _f32),
        pltpu.VMEM((1, _N), _f32),
        pltpu.VMEM((1, 1), _f32),
        pltpu.VMEM((_B, 2), _f32),
    ]

    lat, logit, ro = pl.pallas_call(
        _mega_kernel,
        grid=(_P, _NBLK),
        in_specs=in_specs,
        out_specs=out_specs,
        out_shape=out_shape,
        scratch_shapes=scratch_shapes,
        compiler_params=pltpu.CompilerParams(
            dimension_semantics=("arbitrary", "arbitrary")),
    )(a3, v2, init_w, init_b.reshape(1, _N), gin_eps.reshape(_L, 1, 1),
      wmat, wrow, tr_m1w, tr_m1b[:, None, :], tr_m2w,
      wqkv, bqkv, owsl, cls_w, cls_b[:, None, :])

    return logit, jnp.transpose(lat, (1, 0, 2)), ro.reshape(()) * (1.0 / _G)


# speedup vs baseline: 1.1540x; 1.1540x over previous
import numpy as np
import jax
import jax.numpy as jnp
from jax.experimental import pallas as pl
from jax.experimental.pallas import tpu as pltpu

_B, _T, _N, _H = 4, 64, 111, 111
_L = 4
_HEADS = 3
_HD = _N // _HEADS
_G = _B * _T
_GB = 32
_NBLK = _G // _GB
_RB = _GB * _N
_ROWS = _G * _N
_K = 8624
_EPS_BN = 1e-5
_P = 1 + 3 * _L

_f32 = jnp.float32

_B1, _G1, _BE1, _B2, _G2, _BE2 = 0, 1, 2, 3, 4, 5
_EB, _SG, _SBE, _AB = 6, 7, 8, 9
_OB, _LN1G, _LN1B, _M2B, _LN2G, _LN2B = 10, 11, 12, 13, 14, 15


def _dotT(x, w):
    return jax.lax.dot_general(x, w, (((1,), (1,)), ((), ())),
                               preferred_element_type=_f32)


def _dot(x, w):
    return jax.lax.dot_general(x, w, (((1,), (0,)), ((), ())),
                               preferred_element_type=_f32)


def _mega_kernel(a_ref, v_ref, initw_ref, initb_ref, eps_ref,
                 wmat_ref, wrow_ref, m1w_ref, m1b_ref, m2w_ref,
                 wqkv_ref, bqkv_ref, owsl_ref, cw_ref, cb_ref,
                 lat_ref, log_ref, ro_ref,
                 adj_s, h_s, z_s, xr_s, hm_s,
                 s1_s, q1_s, s2_s, q2_s, ro_s, lg_s):
    p = pl.program_id(0)
    i = pl.program_id(1)
    ph = (p - 1) % 3
    is_g1 = (p >= 1) & (ph == 0)
    is_g2 = (p >= 1) & (ph == 1)
    is_g3 = (p >= 1) & (ph == 2)

    def row(j):
        return wrow_ref[0, j:j + 1, :]

    @pl.when(p == 0)
    def _stage_a():
        a = a_ref[...]
        i32 = jax.lax.bitcast_convert_type(a, jnp.int32)
        sign = jnp.int32(-2147483648)
        mask7 = jnp.int32(0x7FFFFFFF)
        km = jnp.where(i32 < 0, i32 ^ mask7, i32)
        pfx = jnp.zeros((_GB, 1, 1), jnp.int32)
        for bit in range(31, -1, -1):
            bitval = jnp.int32(np.int32(np.uint32(1 << bit)))
            cb = pfx | bitval
            c = cb ^ sign
            lt = (km < c).astype(_f32)
            cnt = jnp.sum(jnp.sum(lt, axis=1, keepdims=True),
                          axis=2, keepdims=True)
            pfx = jnp.where(cnt <= float(_K), cb, pfx)
        key = pfx ^ sign
        thr_i = jnp.where(key < 0, key ^ mask7, key)
        thr = jax.lax.bitcast_convert_type(thr_i, _f32)
        adj_s[i] = (a > thr).astype(_f32)
        h_s[i] = _dotT(v_ref[...], initw_ref[...]) + initb_ref[...]

    @pl.when(is_g1)
    def _g1():
        h = h_s[i]
        parts = []
        for g in range(_GB):
            parts.append(_dot(adj_s[i, g], h[g * _N:(g + 1) * _N, :]))
        agg = jnp.concatenate(parts, axis=0) + eps_ref[0, 0, 0] * h
        z = _dotT(agg, wmat_ref[0, 0]) + row(_B1)
        z_s[i] = z
        s = jnp.sum(z, axis=0, keepdims=True)
        q = jnp.sum(z * z, axis=0, keepdims=True)

        @pl.when(i == 0)
        def _init():
            s1_s[...] = s
            q1_s[...] = q

        @pl.when(i != 0)
        def _acc():
            s1_s[...] += s
            q1_s[...] += q

    @pl.when(is_g2)
    def _g2():
        m = s1_s[...] * (1.0 / _ROWS)
        var = q1_s[...] * (1.0 / _ROWS) - m * m
        sc = row(_G1) / jnp.sqrt(var + _EPS_BN)
        sh = row(_BE1) - m * sc
        y = jnp.maximum(z_s[i] * sc + sh, 0.0)
        z2 = _dotT(y, wmat_ref[0, 1]) + row(_B2)
        z_s[i] = z2
        s = jnp.sum(z2, axis=0, keepdims=True)
        q = jnp.sum(z2 * z2, axis=0, keepdims=True)

        @pl.when(i == 0)
        def _init():
            s2_s[...] = s
            q2_s[...] = q

        @pl.when(i != 0)
        def _acc():
            s2_s[...] += s
            q2_s[...] += q

    @pl.when(is_g3)
    def _g3():
        m = s2_s[...] * (1.0 / _ROWS)
        var = q2_s[...] * (1.0 / _ROWS) - m * m
        sc = row(_G2) / jnp.sqrt(var + _EPS_BN)
        sh = row(_BE2) - m * sc
        h = jnp.maximum(z_s[i] * sc + sh, 0.0)
        h_s[i] = h

        rr = jax.lax.broadcasted_iota(jnp.int32, (_N, _N), 0)
        cc = jax.lax.broadcasted_iota(jnp.int32, (_N, _N), 1)
        upper = cc >= rr
        eye = (cc == rr).astype(_f32)
        ones_row = jnp.ones((1, _N), _f32)

        xrs = []
        hms = []
        ro = jnp.zeros((1, 1), _f32)
        for g in range(_GB):
            hg = h[g * _N:(g + 1) * _N, :]
            xrs.append(_dot(ones_row, hg) * (1.0 / _N))
            hms.append(_dotT(ones_row, hg) * (1.0 / _N))
            mi = _dotT(hg, hg)
            mx = jnp.max(mi, axis=1, keepdims=True)
            mi_n = mi * (1.0 / mx)
            diff = jnp.where(upper, mi_n - eye, 0.0)
            ssq = jnp.sum(jnp.sum(diff * diff, axis=0, keepdims=True),
                          axis=1, keepdims=True)
            ro = ro + jnp.sqrt(ssq)
        xr_s[pl.ds(i * _GB, _GB), :] = jnp.concatenate(xrs, axis=0)
        hm_s[pl.ds(i * _GB, _GB), :] = jnp.concatenate(hms, axis=0)

        first = (p == 3) & (i == 0)

        @pl.when(first)
        def _init():
            ro_s[...] = ro

        @pl.when(jnp.logical_not(first))
        def _acc():
            ro_s[...] += ro

        @pl.when(i == _NBLK - 1)
        def _tail():
            xr = xr_s[...]
            x = _dotT(xr, wmat_ref[0, 2]) + row(_EB)
            mm = jnp.mean(x, axis=0, keepdims=True)
            vv = jnp.mean((x - mm) * (x - mm), axis=0, keepdims=True)
            x = (x - mm) / jnp.sqrt(vv + _EPS_BN) * row(_SG) + row(_SBE)
            x = 0.5 * x * (1.0 + jax.lax.erf(x * np.float32(1.0 / np.sqrt(2.0))))
            gatt = jax.nn.sigmoid(_dotT(x, wmat_ref[0, 3]) + row(_AB))
            hro = gatt * hm_s[...]

            scale = np.float32(1.0 / np.sqrt(_HD))

            def _ln(x, g, b):
                mu = jnp.mean(x, axis=1, keepdims=True)
                va = jnp.mean((x - mu) * (x - mu), axis=1, keepdims=True)
                return (x - mu) / jnp.sqrt(va + _EPS_BN) * g + b

            lgs = []
            for b in range(_B):
                xb = hro[b * _T:(b + 1) * _T, :]
                att = jnp.zeros((_T, _N), _f32)
                for hd in range(_HEADS):
                    q = (_dotT(xb, wqkv_ref[hd, 0]) + bqkv_ref[hd, 0]) * scale
                    kk = _dotT(xb, wqkv_ref[3 + hd, 0]) + bqkv_ref[3 + hd, 0]
                    vvh = _dotT(xb, wqkv_ref[6 + hd, 0]) + bqkv_ref[6 + hd, 0]
                    sc = _dotT(q, kk)
                    mx = jnp.max(sc, axis=1, keepdims=True)
                    e = jnp.exp(sc - mx)
                    pa = e / jnp.sum(e, axis=1, keepdims=True)
                    o = _dot(pa, vvh)
                    att = att + jax.lax.dot_general(
                        o, owsl_ref[hd, 0], (((1,), (1,)), ((), ())),
                        preferred_element_type=_f32)
                att = att + row(_OB)
                x1 = _ln(att, row(_LN1G), row(_LN1B))
                x2 = jnp.maximum(_dotT(x1, m1w_ref[0]) + m1b_ref[0], 0.0)
                x2 = _dotT(x2, m2w_ref[0]) + row(_M2B)
                xo = _ln(x1 + x2, row(_LN2G), row(_LN2B))
                lat = jnp.sum(xo, axis=0, keepdims=True)
                lat_ref[0, b:b + 1, :] = lat
                lgs.append(_dotT(lat, cw_ref[0]) + cb_ref[0])
            lgc = jnp.concatenate(lgs, axis=0)

            @pl.when(p == 3)
            def _lg_init():
                lg_s[...] = lgc

            @pl.when(p != 3)
            def _lg_acc():
                lg_s[...] += lgc

            log_ref[...] = lg_s[...]
            ro_ref[...] = ro_s[...]


def kernel(v, a, init_w, init_b, gin_eps, gin_w1, gin_b1, gin_g1, gin_be1,
           gin_w2, gin_b2, gin_g2, gin_be2, sero_ew, sero_eb, sero_g, sero_be,
           sero_aw, sero_ab, tr_inw, tr_inb, tr_ow, tr_ob, tr_ln1g, tr_ln1b,
           tr_ln2g, tr_ln2b, tr_m1w, tr_m1b, tr_m2w, tr_m2b, cls_w, cls_b):
    a3 = a.reshape(_G, _N, _N)
    v2 = v.reshape(_ROWS, _N)

    wmat = jnp.stack([gin_w1, gin_w2, sero_ew, sero_aw], axis=1)
    wrow = jnp.stack([gin_b1, gin_g1, gin_be1, gin_b2, gin_g2, gin_be2,
                      sero_eb, sero_g, sero_be, sero_ab,
                      tr_ob, tr_ln1g, tr_ln1b, tr_m2b, tr_ln2g, tr_ln2b],
                     axis=1)
    wqkv = jnp.stack([tr_inw[:, base + hd * _HD: base + (hd + 1) * _HD, :]
                      for base in (0, _N, 2 * _N) for hd in range(_HEADS)])
    bqkv = jnp.stack([tr_inb[:, base + hd * _HD: base + (hd + 1) * _HD]
                      for base in (0, _N, 2 * _N)
                      for hd in range(_HEADS)])[:, :, None, :]
    owsl = jnp.stack([tr_ow[:, :, hd * _HD:(hd + 1) * _HD]
                      for hd in range(_HEADS)])

    def _lmap(p, i):
        return jnp.clip((p - 1) // 3, 0, _L - 1)

    def im_av(p, i):
        return (jnp.where(p == 0, i, 0), 0, 0)

    def im_v2(p, i):
        return (jnp.where(p == 0, i, 0), 0)

    def im_const2(p, i):
        return (0, 0)

    def im_l3(p, i):
        return (_lmap(p, i), 0, 0)

    def im_l4(p, i):
        return (0, _lmap(p, i), 0, 0)

    def im_l4a(p, i):
        return (_lmap(p, i), 0, 0, 0)

    in_specs = [
        pl.BlockSpec((_GB, _N, _N), im_av),
        pl.BlockSpec((_RB, _N), im_v2),
        pl.BlockSpec((_N, _N), im_const2),
        pl.BlockSpec((1, _N), im_const2),
        pl.BlockSpec((1, 1, 1), im_l3),
        pl.BlockSpec((1, 4, _N, _N), im_l4a),
        pl.BlockSpec((1, 16, _N), im_l3),
        pl.BlockSpec((1, 2 * _H, _N), im_l3),
        pl.BlockSpec((1, 1, 2 * _H), im_l3),
        pl.BlockSpec((1, _N, 2 * _H), im_l3),
        pl.BlockSpec((9, 1, _HD, _N), im_l4),
        pl.BlockSpec((9, 1, 1, _HD), im_l4),
        pl.BlockSpec((3, 1, _N, _HD), im_l4),
        pl.BlockSpec((1, 2, _N), im_l3),
        pl.BlockSpec((1, 1, 2), im_l3),
    ]
    out_specs = [
        pl.BlockSpec((1, _B, _N), lambda p, i: (_lmap(p, i), 0, 0)),
        pl.BlockSpec((_B, 2), im_const2),
        pl.BlockSpec((1, 1), im_const2),
    ]
    out_shape = [
        jax.ShapeDtypeStruct((_L, _B, _N), _f32),
        jax.ShapeDtypeStruct((_B, 2), _f32),
        jax.ShapeDtypeStruct((1, 1), _f32),
    ]
    scratch_shapes = [
        pltpu.VMEM((_NBLK, _GB, _N, _N), _f32),
        pltpu.VMEM((_NBLK, _RB, _N), _f32),
        pltpu.VMEM((_NBLK, _RB, _N), _f32),
        pltpu.VMEM((_G, _N), _f32),
        pltpu.VMEM((_G, _N), _f32),
        pltpu.VMEM((1, _N), _f32),
        pltpu.VMEM((1, _N), _f32),
        pltpu.VMEM((1, _N), _f32),
        pltpu.VMEM((1, _N), _f32),
        pltpu.VMEM((1, 1), _f32),
        pltpu.VMEM((_B, 2), _f32),
    ]

    lat, logit, ro = pl.pallas_call(
        _mega_kernel,
        grid=(_P, _NBLK),
        in_specs=in_specs,
        out_specs=out_specs,
        out_shape=out_shape,
        scratch_shapes=scratch_shapes,
        compiler_params=pltpu.CompilerParams(
            dimension_semantics=("arbitrary", "arbitrary")),
    )(a3, v2, init_w, init_b.reshape(1, _N), gin_eps.reshape(_L, 1, 1),
      wmat, wrow, tr_m1w, tr_m1b[:, None, :], tr_m2w,
      wqkv, bqkv, owsl, cls_w, cls_b[:, None, :])

    return logit, jnp.transpose(lat, (1, 0, 2)), ro.reshape(()) * (1.0 / _G)


# speedup vs baseline: 1.2031x; 1.0425x over previous
import numpy as np
import jax
import jax.numpy as jnp
from jax.experimental import pallas as pl
from jax.experimental.pallas import tpu as pltpu

_B, _T, _N, _H = 4, 64, 111, 111
_L = 4
_HEADS = 3
_HD = _N // _HEADS
_G = _B * _T
_GB = 64
_NBLK = _G // _GB
_RB = _GB * _N
_ROWS = _G * _N
_K = 8624
_EPS_BN = 1e-5
_P = 1 + 3 * _L

_f32 = jnp.float32

_B1, _G1, _BE1, _B2, _G2, _BE2 = 0, 1, 2, 3, 4, 5
_EB, _SG, _SBE, _AB = 6, 7, 8, 9
_OB, _LN1G, _LN1B, _M2B, _LN2G, _LN2B = 10, 11, 12, 13, 14, 15


def _dotT(x, w):
    return jax.lax.dot_general(x, w, (((1,), (1,)), ((), ())),
                               preferred_element_type=_f32)


def _dot(x, w):
    return jax.lax.dot_general(x, w, (((1,), (0,)), ((), ())),
                               preferred_element_type=_f32)


def _mega_kernel(a_ref, v_ref, initw_ref, initb_ref, eps_ref,
                 wmat_ref, wrow_ref, m1w_ref, m1b_ref, m2w_ref,
                 wqkv_ref, bqkv_ref, owsl_ref, cw_ref, cb_ref,
                 lat_ref, log_ref, ro_ref,
                 thr_s, h_s, z_s, xr_s, hm_s,
                 s1_s, q1_s, s2_s, q2_s, ro_s, lg_s):
    p = pl.program_id(0)
    i = pl.program_id(1)
    ph = (p - 1) % 3
    is_g1 = (p >= 1) & (ph == 0)
    is_g2 = (p >= 1) & (ph == 1)
    is_g3 = (p >= 1) & (ph == 2)

    def row(j):
        return wrow_ref[0, j:j + 1, :]

    @pl.when(p == 0)
    def _stage_a():
        a = a_ref[...]
        i32 = jax.lax.bitcast_convert_type(a, jnp.int32)
        sign = jnp.int32(-2147483648)
        mask7 = jnp.int32(0x7FFFFFFF)
        km = jnp.where(i32 < 0, i32 ^ mask7, i32)
        pfx = jnp.zeros((_GB, 1, 1), jnp.int32)
        for bit in range(31, -1, -1):
            bitval = jnp.int32(np.int32(np.uint32(1 << bit)))
            cb = pfx | bitval
            c = cb ^ sign
            lt = (km < c).astype(_f32)
            cnt = jnp.sum(jnp.sum(lt, axis=1, keepdims=True),
                          axis=2, keepdims=True)
            pfx = jnp.where(cnt <= float(_K), cb, pfx)
        key = pfx ^ sign
        thr_i = jnp.where(key < 0, key ^ mask7, key)
        thr = jax.lax.bitcast_convert_type(thr_i, _f32)
        thr_s[pl.ds(i * _GB, _GB), :] = thr[:, :, 0]
        h_s[i] = _dotT(v_ref[...], initw_ref[...]) + initb_ref[...]

    @pl.when(is_g1)
    def _g1():
        h = h_s[i]
        thrb = thr_s[pl.ds(i * _GB, _GB), :]
        parts = []
        for g in range(_GB):
            adj_g = (a_ref[g] > thrb[g, 0]).astype(_f32)
            parts.append(_dot(adj_g, h[g * _N:(g + 1) * _N, :]))
        agg = jnp.concatenate(parts, axis=0) + eps_ref[0, 0, 0] * h
        z = _dotT(agg, wmat_ref[0, 0]) + row(_B1)
        z_s[i] = z
        s = jnp.sum(z, axis=0, keepdims=True)
        q = jnp.sum(z * z, axis=0, keepdims=True)

        @pl.when(i == 0)
        def _init():
            s1_s[...] = s
            q1_s[...] = q

        @pl.when(i != 0)
        def _acc():
            s1_s[...] += s
            q1_s[...] += q

    @pl.when(is_g2)
    def _g2():
        m = s1_s[...] * (1.0 / _ROWS)
        var = q1_s[...] * (1.0 / _ROWS) - m * m
        sc = row(_G1) / jnp.sqrt(var + _EPS_BN)
        sh = row(_BE1) - m * sc
        y = jnp.maximum(z_s[i] * sc + sh, 0.0)
        z2 = _dotT(y, wmat_ref[0, 1]) + row(_B2)
        z_s[i] = z2
        s = jnp.sum(z2, axis=0, keepdims=True)
        q = jnp.sum(z2 * z2, axis=0, keepdims=True)

        @pl.when(i == 0)
        def _init():
            s2_s[...] = s
            q2_s[...] = q

        @pl.when(i != 0)
        def _acc():
            s2_s[...] += s
            q2_s[...] += q

    @pl.when(is_g3)
    def _g3():
        m = s2_s[...] * (1.0 / _ROWS)
        var = q2_s[...] * (1.0 / _ROWS) - m * m
        sc = row(_G2) / jnp.sqrt(var + _EPS_BN)
        sh = row(_BE2) - m * sc
        h = jnp.maximum(z_s[i] * sc + sh, 0.0)
        h_s[i] = h

        rr = jax.lax.broadcasted_iota(jnp.int32, (_N, _N), 0)
        cc = jax.lax.broadcasted_iota(jnp.int32, (_N, _N), 1)
        upper = cc >= rr
        eye = (cc == rr).astype(_f32)
        ones_row = jnp.ones((1, _N), _f32)

        xrs = []
        hms = []
        ro = jnp.zeros((1, 1), _f32)
        for g in range(_GB):
            hg = h[g * _N:(g + 1) * _N, :]
            xrs.append(_dot(ones_row, hg) * (1.0 / _N))
            hms.append(_dotT(ones_row, hg) * (1.0 / _N))
            mi = _dotT(hg, hg)
            mx = jnp.max(mi, axis=1, keepdims=True)
            mi_n = mi * (1.0 / mx)
            diff = jnp.where(upper, mi_n - eye, 0.0)
            ssq = jnp.sum(jnp.sum(diff * diff, axis=0, keepdims=True),
                          axis=1, keepdims=True)
            ro = ro + jnp.sqrt(ssq)
        xr_s[pl.ds(i * _GB, _GB), :] = jnp.concatenate(xrs, axis=0)
        hm_s[pl.ds(i * _GB, _GB), :] = jnp.concatenate(hms, axis=0)

        first = (p == 3) & (i == 0)

        @pl.when(first)
        def _init():
            ro_s[...] = ro

        @pl.when(jnp.logical_not(first))
        def _acc():
            ro_s[...] += ro

        @pl.when(i == _NBLK - 1)
        def _tail():
            xr = xr_s[...]
            x = _dotT(xr, wmat_ref[0, 2]) + row(_EB)
            mm = jnp.mean(x, axis=0, keepdims=True)
            vv = jnp.mean((x - mm) * (x - mm), axis=0, keepdims=True)
            x = (x - mm) / jnp.sqrt(vv + _EPS_BN) * row(_SG) + row(_SBE)
            x = 0.5 * x * (1.0 + jax.lax.erf(x * np.float32(1.0 / np.sqrt(2.0))))
            gatt = jax.nn.sigmoid(_dotT(x, wmat_ref[0, 3]) + row(_AB))
            hro = gatt * hm_s[...]

            scale = np.float32(1.0 / np.sqrt(_HD))

            def _ln(x, g, b):
                mu = jnp.mean(x, axis=1, keepdims=True)
                va = jnp.mean((x - mu) * (x - mu), axis=1, keepdims=True)
                return (x - mu) / jnp.sqrt(va + _EPS_BN) * g + b

            lgs = []
            for b in range(_B):
                xb = hro[b * _T:(b + 1) * _T, :]
                att = jnp.zeros((_T, _N), _f32)
                for hd in range(_HEADS):
                    q = (_dotT(xb, wqkv_ref[hd, 0]) + bqkv_ref[hd, 0]) * scale
                    kk = _dotT(xb, wqkv_ref[3 + hd, 0]) + bqkv_ref[3 + hd, 0]
                    vvh = _dotT(xb, wqkv_ref[6 + hd, 0]) + bqkv_ref[6 + hd, 0]
                    sc = _dotT(q, kk)
                    mx = jnp.max(sc, axis=1, keepdims=True)
                    e = jnp.exp(sc - mx)
                    pa = e / jnp.sum(e, axis=1, keepdims=True)
                    o = _dot(pa, vvh)
                    att = att + jax.lax.dot_general(
                        o, owsl_ref[hd, 0], (((1,), (1,)), ((), ())),
                        preferred_element_type=_f32)
                att = att + row(_OB)
                x1 = _ln(att, row(_LN1G), row(_LN1B))
                x2 = jnp.maximum(_dotT(x1, m1w_ref[0]) + m1b_ref[0], 0.0)
                x2 = _dotT(x2, m2w_ref[0]) + row(_M2B)
                xo = _ln(x1 + x2, row(_LN2G), row(_LN2B))
                lat = jnp.sum(xo, axis=0, keepdims=True)
                lat_ref[0, b:b + 1, :] = lat
                lgs.append(_dotT(lat, cw_ref[0]) + cb_ref[0])
            lgc = jnp.concatenate(lgs, axis=0)

            @pl.when(p == 3)
            def _lg_init():
                lg_s[...] = lgc

            @pl.when(p != 3)
            def _lg_acc():
                lg_s[...] += lgc

            log_ref[...] = lg_s[...]
            ro_ref[...] = ro_s[...]


def kernel(v, a, init_w, init_b, gin_eps, gin_w1, gin_b1, gin_g1, gin_be1,
           gin_w2, gin_b2, gin_g2, gin_be2, sero_ew, sero_eb, sero_g, sero_be,
           sero_aw, sero_ab, tr_inw, tr_inb, tr_ow, tr_ob, tr_ln1g, tr_ln1b,
           tr_ln2g, tr_ln2b, tr_m1w, tr_m1b, tr_m2w, tr_m2b, cls_w, cls_b):
    a3 = a.reshape(_G, _N, _N)
    v2 = v.reshape(_ROWS, _N)

    wmat = jnp.stack([gin_w1, gin_w2, sero_ew, sero_aw], axis=1)
    wrow = jnp.stack([gin_b1, gin_g1, gin_be1, gin_b2, gin_g2, gin_be2,
                      sero_eb, sero_g, sero_be, sero_ab,
                      tr_ob, tr_ln1g, tr_ln1b, tr_m2b, tr_ln2g, tr_ln2b],
                     axis=1)
    wqkv = jnp.stack([tr_inw[:, base + hd * _HD: base + (hd + 1) * _HD, :]
                      for base in (0, _N, 2 * _N) for hd in range(_HEADS)])
    bqkv = jnp.stack([tr_inb[:, base + hd * _HD: base + (hd + 1) * _HD]
                      for base in (0, _N, 2 * _N)
                      for hd in range(_HEADS)])[:, :, None, :]
    owsl = jnp.stack([tr_ow[:, :, hd * _HD:(hd + 1) * _HD]
                      for hd in range(_HEADS)])

    def _lmap(p, i):
        return jnp.clip((p - 1) // 3, 0, _L - 1)

    def im_av(p, i):
        need = (p == 0) | ((p >= 1) & ((p - 1) % 3 == 0))
        return (jnp.where(need, i, 0), 0, 0)

    def im_v2(p, i):
        return (jnp.where(p == 0, i, 0), 0)

    def im_const2(p, i):
        return (0, 0)

    def im_l3(p, i):
        return (_lmap(p, i), 0, 0)

    def im_l4(p, i):
        return (0, _lmap(p, i), 0, 0)

    def im_l4a(p, i):
        return (_lmap(p, i), 0, 0, 0)

    in_specs = [
        pl.BlockSpec((_GB, _N, _N), im_av),
        pl.BlockSpec((_RB, _N), im_v2),
        pl.BlockSpec((_N, _N), im_const2),
        pl.BlockSpec((1, _N), im_const2),
        pl.BlockSpec((1, 1, 1), im_l3),
        pl.BlockSpec((1, 4, _N, _N), im_l4a),
        pl.BlockSpec((1, 16, _N), im_l3),
        pl.BlockSpec((1, 2 * _H, _N), im_l3),
        pl.BlockSpec((1, 1, 2 * _H), im_l3),
        pl.BlockSpec((1, _N, 2 * _H), im_l3),
        pl.BlockSpec((9, 1, _HD, _N), im_l4),
        pl.BlockSpec((9, 1, 1, _HD), im_l4),
        pl.BlockSpec((3, 1, _N, _HD), im_l4),
        pl.BlockSpec((1, 2, _N), im_l3),
        pl.BlockSpec((1, 1, 2), im_l3),
    ]
    out_specs = [
        pl.BlockSpec((1, _B, _N), lambda p, i: (_lmap(p, i), 0, 0)),
        pl.BlockSpec((_B, 2), im_const2),
        pl.BlockSpec((1, 1), im_const2),
    ]
    out_shape = [
        jax.ShapeDtypeStruct((_L, _B, _N), _f32),
        jax.ShapeDtypeStruct((_B, 2), _f32),
        jax.ShapeDtypeStruct((1, 1), _f32),
    ]
    scratch_shapes = [
        pltpu.VMEM((_G, 1), _f32),
        pltpu.VMEM((_NBLK, _RB, _N), _f32),
        pltpu.VMEM((_NBLK, _RB, _N), _f32),
        pltpu.VMEM((_G, _N), _f32),
        pltpu.VMEM((_G, _N), _f32),
        pltpu.VMEM((1, _N), _f32),
        pltpu.VMEM((1, _N), _f32),
        pltpu.VMEM((1, _N), _f32),
        pltpu.VMEM((1, _N), _f32),
        pltpu.VMEM((1, 1), _f32),
        pltpu.VMEM((_B, 2), _f32),
    ]

    lat, logit, ro = pl.pallas_call(
        _mega_kernel,
        grid=(_P, _NBLK),
        in_specs=in_specs,
        out_specs=out_specs,
        out_shape=out_shape,
        scratch_shapes=scratch_shapes,
        compiler_params=pltpu.CompilerParams(
            dimension_semantics=("arbitrary", "arbitrary")),
    )(a3, v2, init_w, init_b.reshape(1, _N), gin_eps.reshape(_L, 1, 1),
      wmat, wrow, tr_m1w, tr_m1b[:, None, :], tr_m2w,
      wqkv, bqkv, owsl, cls_w, cls_b[:, None, :])

    return logit, jnp.transpose(lat, (1, 0, 2)), ro.reshape(()) * (1.0 / _G)
